# baseline (speedup 1.0000x reference)
import functools

import jax
import jax.numpy as jnp
from jax import lax
from jax.experimental import pallas as pl
from jax.experimental.pallas import tpu as pltpu
from jax.experimental.pallas import tpu_sc as plsc

_INFO = plsc.get_sparse_core_info()
_NC = _INFO.num_cores
_NS = _INFO.num_subcores
_NW = _NC * _NS
_L = _INFO.num_lanes

_B = 16384
_D = 64
_PATHS = 64
_BPW = _B // _NW
_CH = 128
_NCHUNK = _BPW // _CH
_CGROUPS = _CH // _L

_mesh = plsc.VectorSubcoreMesh(core_axis_name="c", subcore_axis_name="s")


@functools.partial(
    pl.kernel,
    out_type=jax.ShapeDtypeStruct((_B,), jnp.float32),
    mesh=_mesh,
    compiler_params=pltpu.CompilerParams(
        needs_layout_passes=False, use_tc_tiling_on_sc=False),
    scratch_types=[
        pltpu.VMEM((_NCHUNK, _CH), jnp.int32),
        pltpu.VMEM((_NCHUNK, _CH), jnp.int32),
        pltpu.VMEM((_BPW,), jnp.int32),
        pltpu.VMEM((2, _CH, _D), jnp.float32),
        pltpu.VMEM((2, _CH, _D), jnp.float32),
        pltpu.VMEM((_PATHS * _D,), jnp.float32),
        pltpu.VMEM((_BPW,), jnp.float32),
        pltpu.SemaphoreType.DMA,
        pltpu.SemaphoreType.DMA,
    ],
)
def _hin2vec_sc(start_hbm, end_hbm, path_hbm, ntab_hbm, ptabf_hbm, out_hbm,
                sidx_v, eidx_v, path_v, srows_v, erows_v, ptab_v, out_v,
                sem0, sem1):
    wid = lax.axis_index("s") * _NC + lax.axis_index("c")
    base = wid * _BPW

    for j in range(_NCHUNK):
        pltpu.sync_copy(start_hbm.at[pl.ds(base + j * _CH, _CH)], sidx_v.at[j])
        pltpu.sync_copy(end_hbm.at[pl.ds(base + j * _CH, _CH)], eidx_v.at[j])
    pltpu.sync_copy(path_hbm.at[pl.ds(base, _BPW)], path_v)
    pltpu.sync_copy(ptabf_hbm, ptab_v)

    sems = (sem0, sem1)

    def fire(j):
        buf = j % 2
        pltpu.async_copy(ntab_hbm.at[sidx_v.at[j]], srows_v.at[buf], sems[buf])
        pltpu.async_copy(ntab_hbm.at[eidx_v.at[j]], erows_v.at[buf], sems[buf])

    def drain(j):
        buf = j % 2
        pltpu.make_async_copy(
            ntab_hbm.at[pl.ds(0, _CH)], srows_v.at[buf], sems[buf]).wait()
        pltpu.make_async_copy(
            ntab_hbm.at[pl.ds(0, _CH)], erows_v.at[buf], sems[buf]).wait()

    lane = lax.broadcasted_iota(jnp.int32, (_L,), 0)

    fire(0)
    for j in range(_NCHUNK):
        drain(j)
        if j + 1 < _NCHUNK:
            fire(j + 1)
        buf = j % 2
        srows = srows_v.at[buf]
        erows = erows_v.at[buf]

        @pl.loop(0, _CGROUPS)
        def group_body(g):
            row_idx = g * _L + lane
            path_g = path_v[pl.ds(j * _CH + g * _L, _L)]
            pathbase = path_g * _D

            @pl.loop(0, _D, init_carry=jnp.zeros((_L,), jnp.float32),
                     unroll=8)
            def dim_body(d, acc):
                dvec = jnp.broadcast_to(d, (_L,)).astype(jnp.int32)
                s_g = plsc.load_gather(srows, [row_idx, dvec])
                e_g = plsc.load_gather(erows, [row_idx, dvec])
                p_g = plsc.load_gather(ptab_v, [pathbase + d])
                return acc + jnp.where(p_g >= 0.0, s_g * e_g, 0.0)

            acc = dim_body
            out_v[pl.ds(j * _CH + g * _L, _L)] = 1.0 / (1.0 + jnp.exp(-acc))

    pltpu.sync_copy(out_v, out_hbm.at[pl.ds(base, _BPW)])


def kernel(start_node, end_node, path, node_table, path_table):
    return _hin2vec_sc(start_node.astype(jnp.int32), end_node.astype(jnp.int32),
                       path.astype(jnp.int32), node_table,
                       path_table.reshape(-1))


# speedup vs baseline: 1.0004x; 1.0004x over previous
import functools

import jax
import jax.numpy as jnp
from jax import lax
from jax.experimental import pallas as pl
from jax.experimental.pallas import tpu as pltpu
from jax.experimental.pallas import tpu_sc as plsc

_INFO = plsc.get_sparse_core_info()
_NC = _INFO.num_cores
_NS = _INFO.num_subcores
_NW = _NC * _NS
_L = _INFO.num_lanes

_B = 16384
_D = 64
_PATHS = 64
_BPW = _B // _NW
_CH = 128
_NCHUNK = _BPW // _CH
_CGROUPS = _CH // _L

_mesh = plsc.VectorSubcoreMesh(core_axis_name="c", subcore_axis_name="s")


@functools.partial(
    pl.kernel,
    out_type=jax.ShapeDtypeStruct((_B,), jnp.float32),
    mesh=_mesh,
    compiler_params=pltpu.CompilerParams(needs_layout_passes=False),
    scratch_types=[
        pltpu.VMEM((_BPW,), jnp.int32),
        pltpu.VMEM((_BPW,), jnp.int32),
        pltpu.VMEM((_BPW,), jnp.int32),
        pltpu.VMEM((_NCHUNK, _CH), jnp.int32),
        pltpu.VMEM((_NCHUNK, _CH), jnp.int32),
        pltpu.VMEM((2, _CH, 2 * _D), jnp.float32),
        pltpu.VMEM((2, _CH, 2 * _D), jnp.float32),
        pltpu.VMEM((_PATHS * _D,), jnp.float32),
        pltpu.VMEM((_BPW,), jnp.float32),
        pltpu.SemaphoreType.DMA,
        pltpu.SemaphoreType.DMA,
    ],
)
def _hin2vec_sc(start_hbm, end_hbm, path_hbm, ntabp_hbm, ptabf_hbm, out_hbm,
                sidx_v, eidx_v, path_v, sblk_v, eblk_v, srows_v, erows_v,
                ptab_v, out_v, sem0, sem1):
    wid = lax.axis_index("s") * _NC + lax.axis_index("c")
    base = wid * _BPW

    pltpu.sync_copy(start_hbm.at[pl.ds(base, _BPW)], sidx_v)
    pltpu.sync_copy(end_hbm.at[pl.ds(base, _BPW)], eidx_v)
    pltpu.sync_copy(path_hbm.at[pl.ds(base, _BPW)], path_v)
    pltpu.sync_copy(ptabf_hbm, ptab_v)

    for j in range(_NCHUNK):
        for g in range(_CH // _L):
            svec = sidx_v[pl.ds(j * _CH + g * _L, _L)]
            evec = eidx_v[pl.ds(j * _CH + g * _L, _L)]
            sblk_v[j, pl.ds(g * _L, _L)] = svec >> 1
            eblk_v[j, pl.ds(g * _L, _L)] = evec >> 1

    sems = (sem0, sem1)

    def fire(j):
        buf = j % 2
        pltpu.async_copy(ntabp_hbm.at[sblk_v.at[j]], srows_v.at[buf],
                         sems[buf])
        pltpu.async_copy(ntabp_hbm.at[eblk_v.at[j]], erows_v.at[buf],
                         sems[buf])

    def drain(j):
        buf = j % 2
        pltpu.make_async_copy(
            ntabp_hbm.at[pl.ds(0, _CH)], srows_v.at[buf], sems[buf]).wait()
        pltpu.make_async_copy(
            ntabp_hbm.at[pl.ds(0, _CH)], erows_v.at[buf], sems[buf]).wait()

    lane = lax.broadcasted_iota(jnp.int32, (_L,), 0)

    fire(0)
    for j in range(_NCHUNK):
        drain(j)
        if j + 1 < _NCHUNK:
            fire(j + 1)
        buf = j % 2
        srows = srows_v.at[buf]
        erows = erows_v.at[buf]

        @pl.loop(0, _CGROUPS)
        def group_body(g):
            row_idx = g * _L + lane
            svec = sidx_v[pl.ds(j * _CH + g * _L, _L)]
            evec = eidx_v[pl.ds(j * _CH + g * _L, _L)]
            scol = (svec & 1) * _D
            ecol = (evec & 1) * _D
            path_g = path_v[pl.ds(j * _CH + g * _L, _L)]
            pathbase = path_g * _D

            @pl.loop(0, _D, init_carry=jnp.zeros((_L,), jnp.float32),
                     unroll=8)
            def dim_body(d, acc):
                s_g = plsc.load_gather(srows, [row_idx, scol + d])
                e_g = plsc.load_gather(erows, [row_idx, ecol + d])
                p_g = plsc.load_gather(ptab_v, [pathbase + d])
                return acc + jnp.where(p_g >= 0.0, s_g * e_g, 0.0)

            acc = dim_body
            out_v[pl.ds(j * _CH + g * _L, _L)] = 1.0 / (1.0 + jnp.exp(-acc))

    pltpu.sync_copy(out_v, out_hbm.at[pl.ds(base, _BPW)])


def kernel(start_node, end_node, path, node_table, path_table):
    ntabp = node_table.reshape(node_table.shape[0] // 2, 2 * _D)
    return _hin2vec_sc(start_node.astype(jnp.int32), end_node.astype(jnp.int32),
                       path.astype(jnp.int32), ntabp,
                       path_table.reshape(-1))
